# Initial kernel scaffold; baseline (speedup 1.0000x reference)
import jax
import jax.numpy as jnp
from jax.experimental import pallas as pl


def kernel(text, offsets, table, W, b):
    raise NotImplementedError("write your pallas kernel here")


# speedup vs baseline: 4.2019x; 4.2019x over previous
import functools

import jax
import jax.numpy as jnp
from jax import lax
from jax.experimental import pallas as pl
from jax.experimental.pallas import tpu as pltpu
from jax.experimental.pallas import tpu_sc as plsc

DIM = 64
L = 50
NC = 2
NS = 16
NW = NC * NS

CB = 16


def _sc_pool_body(text_hbm, table_hbm, out_hbm, idx_v, rows_v, acc_v, sem):
    B = out_hbm.shape[0]
    bags_per_w = B // NW
    n_chunks = bags_per_w // CB
    wid = lax.axis_index("s") * NC + lax.axis_index("c")

    @pl.loop(0, n_chunks)
    def _chunk(c):
        bag_base = wid * bags_per_w + c * CB
        pltpu.sync_copy(text_hbm.at[pl.ds(bag_base, CB)], idx_v)
        descs = []
        for i in range(CB):
            descs.append(
                pltpu.async_copy(
                    table_hbm.at[idx_v.at[i]],
                    rows_v.at[pl.ds(i * L, L)],
                    sem,
                )
            )
        for d in descs:
            d.wait()

        @pl.loop(0, CB)
        def _bag(i):
            base = i * L
            for dc in range(DIM // 16):
                sl = pl.ds(dc * 16, 16)
                acc = rows_v[base, sl]
                for t in range(1, L):
                    acc = acc + rows_v[base + t, sl]
                acc_v[i, sl] = acc

        pltpu.sync_copy(acc_v, out_hbm.at[pl.ds(bag_base, CB)])


def _make_sc_pool(B):
    return pl.kernel(
        _sc_pool_body,
        out_type=jax.ShapeDtypeStruct((B, DIM), jnp.float32),
        mesh=plsc.VectorSubcoreMesh(core_axis_name="c", subcore_axis_name="s"),
        scratch_types=[
            pltpu.VMEM((CB, L), jnp.int32),
            pltpu.VMEM((CB * L, DIM), jnp.float32),
            pltpu.VMEM((CB, DIM), jnp.float32),
            pltpu.SemaphoreType.DMA,
        ],
        compiler_params=pltpu.CompilerParams(use_tc_tiling_on_sc=False),
    )


def _head_body(sums_ref, w_ref, b_ref, out_ref):
    x = sums_ref[...] * (1.0 / L)
    logits = jnp.dot(x, w_ref[...], preferred_element_type=jnp.float32)
    logits = logits + b_ref[...]
    m = jnp.max(logits, axis=1, keepdims=True)
    e = jnp.exp(logits - m)
    out_ref[...] = e / jnp.sum(e, axis=1, keepdims=True)


def _head(sums, W, b):
    B = sums.shape[0]
    C = W.shape[1]
    blk = 512
    grid = (B // blk,)
    return pl.pallas_call(
        _head_body,
        grid=grid,
        in_specs=[
            pl.BlockSpec((blk, DIM), lambda i: (i, 0)),
            pl.BlockSpec((DIM, C), lambda i: (0, 0)),
            pl.BlockSpec((1, C), lambda i: (0, 0)),
        ],
        out_specs=pl.BlockSpec((blk, C), lambda i: (i, 0)),
        out_shape=jax.ShapeDtypeStruct((B, C), jnp.float32),
    )(sums, W, b.reshape(1, C))


def kernel(text, offsets, table, W, b):
    B = offsets.shape[0]
    text2d = text.reshape(B, L)
    sums = _make_sc_pool(B)(text2d, table)
    return _head(sums, W, b)


# speedup vs baseline: 4.2037x; 1.0004x over previous
import functools

import jax
import jax.numpy as jnp
from jax import lax
from jax.experimental import pallas as pl
from jax.experimental.pallas import tpu as pltpu
from jax.experimental.pallas import tpu_sc as plsc

DIM = 64
L = 50
NC = 2
NS = 16
NW = NC * NS

CB = 16


def _sc_pool_body(text_hbm, table_hbm, out_hbm, idx_v, rows_v, acc_v, sem):
    B = out_hbm.shape[0]
    bags_per_w = B // NW
    n_chunks = bags_per_w // CB
    wid = lax.axis_index("s") * NC + lax.axis_index("c")

    @pl.loop(0, n_chunks)
    def _chunk(c):
        bag_base = wid * bags_per_w + c * CB
        pltpu.sync_copy(text_hbm.at[pl.ds(bag_base, CB)], idx_v)
        descs = []
        for i in range(CB):
            descs.append(
                pltpu.async_copy(
                    table_hbm.at[idx_v.at[i]],
                    rows_v.at[pl.ds(i * L, L)],
                    sem,
                )
            )
        for d in descs:
            d.wait()

        @pl.loop(0, CB)
        def _bag(i):
            base = i * L
            for dc in range(DIM // 16):
                sl = pl.ds(dc * 16, 16)
                acc = rows_v[base, sl]
                for t in range(1, L):
                    acc = acc + rows_v[base + t, sl]
                acc_v[i, sl] = acc

        pltpu.sync_copy(acc_v, out_hbm.at[pl.ds(bag_base, CB)])


def _make_sc_pool(B):
    return pl.kernel(
        _sc_pool_body,
        out_type=jax.ShapeDtypeStruct((B, DIM), jnp.float32),
        mesh=plsc.VectorSubcoreMesh(core_axis_name="c", subcore_axis_name="s"),
        scratch_types=[
            pltpu.VMEM((CB, L), jnp.int32),
            pltpu.VMEM((CB * L, DIM), jnp.float32),
            pltpu.VMEM((CB, DIM), jnp.float32),
            pltpu.SemaphoreType.DMA,
        ],
        compiler_params=pltpu.CompilerParams(use_tc_tiling_on_sc=False),
    )


def _head_body(sums_ref, w_ref, b_ref, out_ref):
    x = sums_ref[...] * (1.0 / L)
    logits = jnp.dot(x, w_ref[...], preferred_element_type=jnp.float32)
    logits = logits + b_ref[...]
    m = jnp.max(logits, axis=1, keepdims=True)
    e = jnp.exp(logits - m)
    out_ref[...] = e / jnp.sum(e, axis=1, keepdims=True)


def _head(sums, W, b):
    B = sums.shape[0]
    C = W.shape[1]
    blk = 512
    grid = (B // blk,)
    return pl.pallas_call(
        _head_body,
        grid=grid,
        in_specs=[
            pl.BlockSpec((blk, DIM), lambda i: (i, 0)),
            pl.BlockSpec((DIM, C), lambda i: (0, 0)),
            pl.BlockSpec((1, C), lambda i: (0, 0)),
        ],
        out_specs=pl.BlockSpec((blk, C), lambda i: (i, 0)),
        out_shape=jax.ShapeDtypeStruct((B, C), jnp.float32),
    )(sums, W, b.reshape(1, C))


def kernel(text, offsets, table, W, b):
    B = offsets.shape[0]
    text2d = text.reshape(B, L)
    V = table.shape[0]
    packed = jax.lax.optimization_barrier(table.reshape(V // 2, 2 * DIM))
    table_lin = packed.reshape(V, DIM)
    sums = _make_sc_pool(B)(text2d, table_lin)
    return _head(sums, W, b)


# speedup vs baseline: 4.9049x; 1.1668x over previous
import functools

import jax
import jax.numpy as jnp
from jax import lax
from jax.experimental import pallas as pl
from jax.experimental.pallas import tpu as pltpu
from jax.experimental.pallas import tpu_sc as plsc

DIM = 64
L = 50
NC = 2
NS = 16
NW = NC * NS

CB = 16


def _sc_pool_body(text_hbm, table_hbm, out_hbm, idx_v, rows_v, acc_v, sem):
    B = out_hbm.shape[0]
    bags_per_w = B // NW
    n_chunks = bags_per_w // CB
    wid = lax.axis_index("s") * NC + lax.axis_index("c")

    @pl.loop(0, n_chunks)
    def _chunk(c):
        bag_base = wid * bags_per_w + c * CB
        pltpu.sync_copy(text_hbm.at[pl.ds(bag_base, CB)], idx_v)
        descs = []
        for i in range(CB):
            descs.append(
                pltpu.async_copy(
                    table_hbm.at[idx_v.at[i]],
                    rows_v.at[pl.ds(i * L, L)],
                    sem,
                )
            )
        for d in descs:
            d.wait()

        @pl.loop(0, CB)
        def _bag(i):
            base = i * L
            for dc in range(DIM // 16):
                sl = pl.ds(dc * 16, 16)
                acc = rows_v[base, sl]
                for t in range(1, L):
                    acc = acc + rows_v[base + t, sl]
                acc_v[i, sl] = acc

        pltpu.sync_copy(acc_v, out_hbm.at[pl.ds(bag_base, CB)])


def _make_sc_pool(B):
    return pl.kernel(
        _sc_pool_body,
        out_type=jax.ShapeDtypeStruct((B, DIM), jnp.float32),
        mesh=plsc.VectorSubcoreMesh(core_axis_name="c", subcore_axis_name="s"),
        scratch_types=[
            pltpu.VMEM((CB, L), jnp.int32),
            pltpu.VMEM((CB * L, DIM), jnp.float32),
            pltpu.VMEM((CB, DIM), jnp.float32),
            pltpu.SemaphoreType.DMA,
        ],
        compiler_params=pltpu.CompilerParams(use_tc_tiling_on_sc=False),
    )


_RBL = 2048


def _repack_body(x_ref, out_ref):
    x = x_ref[...]
    xt = x.T
    out_ref[...] = jnp.concatenate([xt[: _RBL // 2], xt[_RBL // 2:]], axis=1)


def _repack(tableT):
    V = tableT.shape[1]
    G = pl.cdiv(V, _RBL)
    return pl.pallas_call(
        _repack_body,
        grid=(G,),
        in_specs=[pl.BlockSpec((DIM, _RBL), lambda g: (0, g))],
        out_specs=pl.BlockSpec((_RBL // 2, 2 * DIM), lambda g: (g, 0)),
        out_shape=jax.ShapeDtypeStruct((G * _RBL // 2, 2 * DIM), jnp.float32),
    )(tableT)


def _head_body(sums_ref, w_ref, b_ref, out_ref):
    x = sums_ref[...] * (1.0 / L)
    logits = jnp.dot(x, w_ref[...], preferred_element_type=jnp.float32)
    logits = logits + b_ref[...]
    m = jnp.max(logits, axis=1, keepdims=True)
    e = jnp.exp(logits - m)
    out_ref[...] = e / jnp.sum(e, axis=1, keepdims=True)


def _head(sums, W, b):
    B = sums.shape[0]
    C = W.shape[1]
    blk = 512
    grid = (B // blk,)
    return pl.pallas_call(
        _head_body,
        grid=grid,
        in_specs=[
            pl.BlockSpec((blk, DIM), lambda i: (i, 0)),
            pl.BlockSpec((DIM, C), lambda i: (0, 0)),
            pl.BlockSpec((1, C), lambda i: (0, 0)),
        ],
        out_specs=pl.BlockSpec((blk, C), lambda i: (i, 0)),
        out_shape=jax.ShapeDtypeStruct((B, C), jnp.float32),
    )(sums, W, b.reshape(1, C))


def kernel(text, offsets, table, W, b):
    B = offsets.shape[0]
    packed = _repack(table.T)
    table_lin = packed.reshape(packed.shape[0] * 2, DIM)
    h = _RBL // 2
    textq = ((text >> 11) << 11) + ((text & (h - 1)) << 1) + ((text >> 10) & 1)
    text2d = textq.reshape(B, L)
    sums = _make_sc_pool(B)(text2d, table_lin)
    return _head(sums, W, b)
